# Initial kernel scaffold; baseline (speedup 1.0000x reference)
import jax
import jax.numpy as jnp
from jax.experimental import pallas as pl


def kernel(x, edge_index, edge_attr, batch, W_emb, b_emb, W1_0, b1_0, W2_0, W3_0, b3_0, W1_1, b1_1, W2_1, W3_1, b3_1, Wc1, bc1, Wc2, bc2):
    raise NotImplementedError("write your pallas kernel here")


# speedup vs baseline: 11.7468x; 11.7468x over previous
import functools

import jax
import jax.numpy as jnp
from jax import lax
from jax.experimental import pallas as pl
from jax.experimental.pallas import tpu as pltpu
from jax.experimental.pallas import tpu_sc as plsc

N = 10000
NPAD = 10240
E = 320000
HID = 128
NG = 64

_UNPACK_PERM = []
for _k in range(4):
    for _m in range(16):
        _UNPACK_PERM.append(32 * _k + _m)
        _UNPACK_PERM.append(32 * _k + 16 + _m)

NC = 2
NS = 16
NW = NC * NS
EW = E // NW
CHUNK = 40
NCHUNK = EW // CHUNK
ROWS_PER_TILE = NPAD // NS

MB = 512
NBLK = NPAD // MB


def _sc_body(with_deg, *refs):
    if with_deg:
        (bb_hbm, sd_hbm, ea_hbm, s_hbm, deg_hbm,
         acc, accd, sd_v, ea_v, i0, i1, i2, o0, o1, o2,
         sc0, sc1, sc2, dc0, dc1, dc2, zvec,
         g0, g1, g2, w0, w1, w2) = refs
    else:
        (bb_hbm, sd_hbm, ea_hbm, s_hbm,
         acc, sd_v, ea_v, i0, i1, i2, o0, o1, o2,
         sc0, sc1, sc2, dc0, dc1, dc2,
         g0, g1, g2, w0, w1, w2) = refs
        deg_hbm = accd = zvec = None
    inb = (i0, i1, i2)
    outb = (o0, o1, o2)
    srcc = (sc0, sc1, sc2)
    dstc = (dc0, dc1, dc2)
    gsems = (g0, g1, g2)
    wsems = (w0, w1, w2)
    rows_v = o0
    cid = lax.axis_index("c")
    sid = lax.axis_index("s")

    zeros16 = jnp.zeros((16,), jnp.float32)

    def zb(i, _):
        for k in range(8):
            rows_v[i, pl.ds(k * 16, 16)] = zeros16
        return 0
    lax.fori_loop(0, CHUNK, zb, 0)

    if with_deg:
        def zv(i, _):
            zvec[pl.ds(i * 16, 16)] = zeros16
            return 0
        lax.fori_loop(0, ROWS_PER_TILE // 16, zv, 0)

    def zacc(j, _):
        pltpu.sync_copy(rows_v,
                        acc.at[pl.ds(sid * ROWS_PER_TILE + j * CHUNK, CHUNK)])
        return 0
    lax.fori_loop(0, ROWS_PER_TILE // CHUNK, zacc, 0)
    if with_deg:
        pltpu.sync_copy(zvec, accd.at[pl.ds(sid * ROWS_PER_TILE, ROWS_PER_TILE)])

    pltpu.sync_copy(sd_hbm.at[cid, sid], sd_v)
    pltpu.sync_copy(ea_hbm.at[cid, sid], ea_v)

    def unpack_idx(t, buf, which):
        for off in (0, 16, 24):
            p16 = sd_v[t, pl.ds(off, 16)]
            if which == 0:
                buf[pl.ds(off, 16)] = lax.bitwise_and(p16, 16383)
            else:
                buf[pl.ds(off, 16)] = lax.shift_right_logical(p16, 14)

    def unpack_src(t, b):
        unpack_idx(t, srcc[b], 0)

    def unpack_dst(t, b):
        unpack_idx(t, dstc[b], 1)

    plsc.subcore_barrier()

    def start_gather(t, b):
        pltpu.async_copy(bb_hbm.at[srcc[b].at[pl.ds(0, 24)]],
                         inb[b].at[pl.ds(0, 24)], gsems[b])
        pltpu.async_copy(bb_hbm.at[srcc[b].at[pl.ds(24, 16)]],
                         inb[b].at[pl.ds(24, 16)], gsems[b])

    def wait_gather(t, b):
        pltpu.make_async_copy(bb_hbm.at[srcc[b].at[pl.ds(0, 24)]],
                              inb[b].at[pl.ds(0, 24)], gsems[b]).wait()
        pltpu.make_async_copy(bb_hbm.at[srcc[b].at[pl.ds(24, 16)]],
                              inb[b].at[pl.ds(24, 16)], gsems[b]).wait()

    def start_scatter(t, b):
        pltpu.async_copy(outb[b], acc.at[dstc[b]], wsems[b], add=True)
        if with_deg:
            pltpu.sync_copy(ea_v.at[t], accd.at[dstc[b]], add=True)

    def wait_scatter(t, b):
        pltpu.make_async_copy(outb[b], acc.at[dstc[b]], wsems[b]).wait()

    _DN = lax.GatherDimensionNumbers(offset_dims=(), collapsed_slice_dims=(0,),
                                     start_index_map=(0,))

    def scale(t, b):
        rin = inb[b]
        rout = outb[b]
        for base, lo in ((0, 0), (16, 0), (24, 8)):
            w16 = ea_v[t, pl.ds(base, 16)]

            @plsc.parallel_loop(lo, 16, 1, unroll=4)
            def _(j, w16=w16, base=base):
                w = lax.gather(w16, jnp.full((16, 1), j, jnp.int32), _DN, (1,),
                               mode=lax.GatherScatterMode.PROMISE_IN_BOUNDS)
                i = base + j
                for k in range(4):
                    ab = rin[i, pl.ds(k * 32, 32)]
                    lo16, hi16 = plsc.unpack(ab, format=plsc.PackFormat.INTERLEAVED)
                    rout[i, pl.ds(k * 32, 16)] = lo16 * w
                    rout[i, pl.ds(k * 32 + 16, 16)] = hi16 * w

    unpack_src(0, 0)
    start_gather(0, 0)
    unpack_src(1, 1)
    start_gather(1, 1)

    def triple(g, _):
        for j in range(3):
            t = 3 * g + j

            @pl.when(t < NCHUNK)
            def _():
                wait_gather(t, j)
                unpack_dst(t, j)
                scale(t, j)
                start_scatter(t, j)

                @pl.when(t + 2 < NCHUNK)
                def _():
                    @pl.when(t >= 1)
                    def _():
                        wait_scatter(t - 1, (j + 2) % 3)
                    unpack_src(t + 2, (j + 2) % 3)
                    start_gather(t + 2, (j + 2) % 3)
        return 0
    lax.fori_loop(0, (NCHUNK + 2) // 3, triple, 0)

    for t in range(max(NCHUNK - 3, 0), NCHUNK):
        wait_scatter(t, t % 3)

    plsc.subcore_barrier()

    r0 = sid * ROWS_PER_TILE
    pltpu.sync_copy(acc.at[pl.ds(r0, ROWS_PER_TILE)],
                    s_hbm.at[cid, pl.ds(r0, ROWS_PER_TILE)])
    if with_deg:
        pltpu.sync_copy(accd.at[pl.ds(r0, ROWS_PER_TILE)],
                        deg_hbm.at[cid, pl.ds(r0, ROWS_PER_TILE)])


@functools.lru_cache(maxsize=None)
def _make_sc_scatter(with_deg):
    mesh = plsc.VectorSubcoreMesh(core_axis_name="c", subcore_axis_name="s",
                                  num_cores=NC, num_subcores=NS)
    out_type = [jax.ShapeDtypeStruct((NC, NPAD, HID), jnp.float32)]
    if with_deg:
        out_type.append(jax.ShapeDtypeStruct((NC, NPAD), jnp.float32))
    scratch = [pltpu.VMEM_SHARED((NPAD, HID), jnp.float32)]
    if with_deg:
        scratch.append(pltpu.VMEM_SHARED((NPAD,), jnp.float32))
    scratch += [
        pltpu.VMEM((NCHUNK, CHUNK), jnp.int32),
        pltpu.VMEM((NCHUNK, CHUNK), jnp.float32),
        pltpu.VMEM((CHUNK, HID), jnp.bfloat16),
        pltpu.VMEM((CHUNK, HID), jnp.bfloat16),
        pltpu.VMEM((CHUNK, HID), jnp.bfloat16),
        pltpu.VMEM((CHUNK, HID), jnp.float32),
        pltpu.VMEM((CHUNK, HID), jnp.float32),
        pltpu.VMEM((CHUNK, HID), jnp.float32),
        pltpu.VMEM((CHUNK,), jnp.int32),
        pltpu.VMEM((CHUNK,), jnp.int32),
        pltpu.VMEM((CHUNK,), jnp.int32),
        pltpu.VMEM((CHUNK,), jnp.int32),
        pltpu.VMEM((CHUNK,), jnp.int32),
        pltpu.VMEM((CHUNK,), jnp.int32),
    ]
    if with_deg:
        scratch.append(pltpu.VMEM((ROWS_PER_TILE,), jnp.float32))
    scratch += [pltpu.SemaphoreType.DMA] * 6
    return pl.kernel(
        functools.partial(_sc_body, with_deg),
        out_type=tuple(out_type) if with_deg else out_type[0],
        mesh=mesh,
        scratch_types=scratch,
        compiler_params=pltpu.CompilerParams(needs_layout_passes=False,
                                             use_tc_tiling_on_sc=False),
        name=("sc_scatter_deg" if with_deg else "sc_scatter"),
    )


def _colbcast(row):
    return lax.dot_general(row, jnp.ones((1, 1), jnp.float32),
                           (((0,), (0,)), ((), ())),
                           preferred_element_type=jnp.float32)


def _k1a_body(x_ref, wemb_ref, bemb_ref, w2_ref, bb_ref):
    h = jnp.dot(x_ref[...], wemb_ref[...],
                preferred_element_type=jnp.float32) + bemb_ref[...]
    bb_ref[...] = jnp.dot(h, w2_ref[...],
                          preferred_element_type=jnp.float32).astype(jnp.bfloat16)


def _k2a_body(x_ref, s0_ref, s1_ref, d0_ref, d1_ref,
              wemb_ref, bemb_ref, w1_ref, b1_ref, w3_ref, b3_ref, w2_ref,
              bb2_ref):
    a0, s0 = _dense(x_ref[...], wemb_ref[...], bemb_ref[...], w1_ref[...],
                    b1_ref[...], w3_ref[...], b3_ref[...])
    h1 = _combine(a0, s0, s0_ref, s1_ref, d0_ref, d1_ref)
    bb2_ref[...] = jnp.dot(h1, w2_ref[...],
                           preferred_element_type=jnp.float32).astype(jnp.bfloat16)


def _dense(x, wemb, bemb, w1, b1, w3, b3):
    h = jnp.dot(x, wemb, preferred_element_type=jnp.float32) + bemb
    a = jnp.dot(h, w1, preferred_element_type=jnp.float32) + b1
    sv = jnp.dot(h, w3, preferred_element_type=jnp.float32) + b3
    return a, sv


def _combine(a, sv, s0_ref, s1_ref, d0_ref, d1_ref):
    deg = d0_ref[0] + d1_ref[0]
    degc = _colbcast(deg)
    return jnp.maximum(a * degc - s0_ref[...] - s1_ref[...] + sv, 0.0)


def _k3_body(x_ref, s0_ref, s1_ref, t0_ref, t1_ref, d0_ref, d1_ref, batch_ref,
             wemb_ref, bemb_ref, w10_ref, b10_ref, w30_ref, b30_ref,
             w11_ref, b11_ref, w31_ref, b31_ref,
             wc1_ref, bc1_ref, wc2_ref, bc2_ref,
             pred_ref, acc_ref, cnt_ref):
    i = pl.program_id(0)

    @pl.when(i == 0)
    def _():
        acc_ref[...] = jnp.zeros_like(acc_ref)
        cnt_ref[...] = jnp.zeros_like(cnt_ref)

    a0, s0 = _dense(x_ref[...], wemb_ref[...], bemb_ref[...], w10_ref[...],
                    b10_ref[...], w30_ref[...], b30_ref[...])
    h1 = _combine(a0, s0, s0_ref, s1_ref, d0_ref, d1_ref)
    a1 = jnp.dot(h1, w11_ref[...], preferred_element_type=jnp.float32) + b11_ref[...]
    s1v = jnp.dot(h1, w31_ref[...], preferred_element_type=jnp.float32) + b31_ref[...]
    h = _combine(a1, s1v, t0_ref, t1_ref, d0_ref, d1_ref)

    b = batch_ref[0].astype(jnp.float32)
    bcol = _colbcast(b)
    iota = lax.broadcasted_iota(jnp.int32, (MB, 128), 1).astype(jnp.float32)
    onehot = jnp.where(bcol == iota, 1.0, 0.0)

    acc_ref[...] += lax.dot_general(onehot, h, (((0,), (0,)), ((), ())),
                                    preferred_element_type=jnp.float32)
    cnt_ref[...] += jnp.dot(jnp.ones((1, MB), jnp.float32), onehot,
                            preferred_element_type=jnp.float32)

    @pl.when(i == NBLK - 1)
    def _():
        cntc = _colbcast(cnt_ref[...])
        gx = acc_ref[...] * (1.0 / jnp.maximum(cntc, 1.0))
        z = jnp.maximum(
            jnp.dot(gx, wc1_ref[...],
                    preferred_element_type=jnp.float32) + bc1_ref[...], 0.0)
        pred_ref[...] = jnp.dot(z, wc2_ref[...],
                                preferred_element_type=jnp.float32) + bc2_ref[...]


_W_SPEC = pl.BlockSpec((HID, HID), lambda i: (0, 0))
_B_SPEC = pl.BlockSpec((1, HID), lambda i: (0, 0))
_ROW_SPEC = pl.BlockSpec((MB, HID), lambda i: (i, 0))
_D_SPEC = pl.BlockSpec((1, 1, MB), lambda i: (i, 0, 0))


def _k1a(x, wembT, bemb, w2T):
    outh = jax.ShapeDtypeStruct((NPAD, HID), jnp.bfloat16)
    return pl.pallas_call(
        _k1a_body,
        grid=(NBLK,),
        in_specs=[_ROW_SPEC, _W_SPEC, _B_SPEC, _W_SPEC],
        out_specs=_ROW_SPEC,
        out_shape=outh,
    )(x, wembT, bemb, w2T)


def _k2a(x, s0, s1, d0, d1, wembT, bemb, w1T, b1, w3T, b3, w2T):
    outh = jax.ShapeDtypeStruct((NPAD, HID), jnp.bfloat16)
    return pl.pallas_call(
        _k2a_body,
        grid=(NBLK,),
        in_specs=[_ROW_SPEC, _ROW_SPEC, _ROW_SPEC, _D_SPEC, _D_SPEC,
                  _W_SPEC, _B_SPEC, _W_SPEC, _B_SPEC, _W_SPEC, _B_SPEC,
                  _W_SPEC],
        out_specs=_ROW_SPEC,
        out_shape=outh,
    )(x, s0, s1, d0, d1, wembT, bemb, w1T, b1, w3T, b3, w2T)


def _k3(x, s0, s1, t0, t1, d0, d1, batch3, wembT, bemb, w10T, b10, w30T, b30,
        w11T, b11, w31T, b31, wc1T, bc1, wc2Tp, bc2p):
    return pl.pallas_call(
        _k3_body,
        grid=(NBLK,),
        in_specs=[_ROW_SPEC, _ROW_SPEC, _ROW_SPEC, _ROW_SPEC, _ROW_SPEC,
                  _D_SPEC, _D_SPEC,
                  pl.BlockSpec((1, 1, MB), lambda i: (i, 0, 0)),
                  _W_SPEC, _B_SPEC, _W_SPEC, _B_SPEC, _W_SPEC, _B_SPEC,
                  _W_SPEC, _B_SPEC, _W_SPEC, _B_SPEC,
                  pl.BlockSpec((HID, 2 * HID), lambda i: (0, 0)),
                  pl.BlockSpec((1, 2 * HID), lambda i: (0, 0)),
                  pl.BlockSpec((2 * HID, HID), lambda i: (0, 0)),
                  _B_SPEC],
        out_specs=pl.BlockSpec((128, HID), lambda i: (0, 0)),
        out_shape=jax.ShapeDtypeStruct((128, HID), jnp.float32),
        scratch_shapes=[pltpu.VMEM((128, HID), jnp.float32),
                        pltpu.VMEM((1, 128), jnp.float32)],
    )(x, s0, s1, t0, t1, d0, d1, batch3, wembT, bemb, w10T, b10, w30T, b30,
      w11T, b11, w31T, b31, wc1T, bc1, wc2Tp, bc2p)


@jax.jit
def kernel(x, edge_index, edge_attr, batch, W_emb, b_emb,
           W1_0, b1_0, W2_0, W3_0, b3_0,
           W1_1, b1_1, W2_1, W3_1, b3_1,
           Wc1, bc1, Wc2, bc2):
    xp = jnp.pad(x, ((0, NPAD - N), (0, 0)))
    batch3 = jnp.pad(batch, (0, NPAD - N), constant_values=NG).reshape(
        NBLK, 1, MB)
    packed = (edge_index[1] * 16384 + edge_index[0]).reshape(
        NC, NS, NCHUNK, CHUNK)
    ea = edge_attr.reshape(NC, NS, NCHUNK, CHUNK)
    qcols = jnp.asarray(_UNPACK_PERM, dtype=jnp.int32)

    r2 = lambda v: v.reshape(1, -1)
    bb0 = _k1a(xp, W_emb.T, r2(b_emb), W2_0.T[:, qcols])
    S0, deg = _make_sc_scatter(True)(bb0, packed, ea)
    d0 = deg[0].reshape(NBLK, 1, MB)
    d1 = deg[1].reshape(NBLK, 1, MB)

    bb1 = _k2a(xp, S0[0], S0[1], d0, d1, W_emb.T, r2(b_emb), W1_0.T,
               r2(b1_0), W3_0.T, r2(b3_0), W2_1.T[:, qcols])
    S1 = _make_sc_scatter(False)(bb1, packed, ea)

    wc2Tp = jnp.pad(Wc2.T, ((0, 0), (0, HID - Wc2.shape[0])))
    bc2p = jnp.pad(r2(bc2), ((0, 0), (0, HID - Wc2.shape[0])))
    pred128 = _k3(xp, S0[0], S0[1], S1[0], S1[1], d0, d1, batch3,
                  W_emb.T, r2(b_emb), W1_0.T, r2(b1_0), W3_0.T, r2(b3_0),
                  W1_1.T, r2(b1_1), W3_1.T, r2(b3_1),
                  Wc1.T, r2(bc1), wc2Tp, bc2p)
    return pred128[:NG, :Wc2.shape[0]]
